# baseline (speedup 1.0000x reference)
import functools
import math

import jax
import jax.numpy as jnp
from jax import lax
from jax.experimental import pallas as pl
from jax.experimental.pallas import tpu as pltpu
from jax.experimental.pallas import tpu_sc as plsc

N = 100000
E = 1600000
NODE_DIM = 128
NUM_BASIS = 20
CUTOFF = 5.0

_NC = 2
_NS = 16
_NW = _NC * _NS

_BPW = 3128
_CHUNK = 512
_NCHUNK = 7
_N_PAD = _BPW * _NW


def _sc_gather_kernel(idx_hbm, table_hbm, out_hbm, idx_v, rows_v, sem):
    wid = lax.axis_index("s") * _NC + lax.axis_index("c")
    base = wid * _BPW
    count = jnp.where(wid == _NW - 1, N - (_NW - 1) * _BPW, _BPW)
    pltpu.sync_copy(idx_hbm.at[pl.ds(base, _BPW)], idx_v)
    for i in range(_NCHUNK):
        s = jnp.minimum(i * _CHUNK, count - _CHUNK)
        pltpu.async_copy(
            table_hbm.at[idx_v.at[pl.ds(s, _CHUNK)]], rows_v, sem
        ).wait()
        pltpu.sync_copy(rows_v, out_hbm.at[pl.ds(base + s, _CHUNK)])


def _sc_gather(atomic_numbers, embed_table):
    idx = jnp.pad(atomic_numbers.astype(jnp.int32), (0, _N_PAD - N))
    mesh = plsc.VectorSubcoreMesh(core_axis_name="c", subcore_axis_name="s")
    kern = functools.partial(
        pl.kernel,
        mesh=mesh,
        out_type=jax.ShapeDtypeStruct((N, NODE_DIM), jnp.float32),
        scratch_types=[
            pltpu.VMEM((_BPW,), jnp.int32),
            pltpu.VMEM((_CHUNK, NODE_DIM), jnp.float32),
            pltpu.SemaphoreType.DMA,
        ],
    )(_sc_gather_kernel)
    return kern(idx, embed_table)


_BE = 6400


def _tc_edge_kernel(d_ref, ev_ref, rbf_ref, fcut_ref, uvec_ref):
    d = d_ref[...]
    theta = d * (math.pi / CUTOFF)
    n = (
        lax.broadcasted_iota(jnp.int32, (_BE, NUM_BASIS), 1) + 1
    ).astype(jnp.float32)
    coef = math.sqrt(2.0 / CUTOFF)
    rbf_ref[...] = coef * jnp.sin(n * theta) / d
    fcut_ref[...] = 0.5 * (jnp.cos(theta) + 1.0) * (d < CUTOFF).astype(
        jnp.float32
    )
    uvec_ref[...] = ev_ref[...] / d


def _tc_edges(edge_vector, edge_length):
    d2 = edge_length.reshape(E, 1)
    grid = E // _BE
    return pl.pallas_call(
        _tc_edge_kernel,
        grid=(grid,),
        in_specs=[
            pl.BlockSpec((_BE, 1), lambda i: (i, 0)),
            pl.BlockSpec((_BE, 3), lambda i: (i, 0)),
        ],
        out_specs=[
            pl.BlockSpec((_BE, NUM_BASIS), lambda i: (i, 0)),
            pl.BlockSpec((_BE, 1), lambda i: (i, 0)),
            pl.BlockSpec((_BE, 3), lambda i: (i, 0)),
        ],
        out_shape=[
            jax.ShapeDtypeStruct((E, NUM_BASIS), jnp.float32),
            jax.ShapeDtypeStruct((E, 1), jnp.float32),
            jax.ShapeDtypeStruct((E, 3), jnp.float32),
        ],
    )(d2, edge_vector)


def kernel(atomic_numbers, edge_vector, edge_length, embed_table):
    node_invariant = _sc_gather(atomic_numbers, embed_table)
    rbf, fcut, uvec = _tc_edges(edge_vector, edge_length)
    node_equivariant = jnp.zeros((N, 3, NODE_DIM), dtype=jnp.float32)
    return (node_invariant, rbf, fcut, uvec, node_equivariant)
